# baseline (speedup 1.0000x reference)
import functools

import jax
import jax.numpy as jnp
from jax import lax
from jax.experimental import pallas as pl
from jax.experimental.pallas import tpu as pltpu
from jax.experimental.pallas import tpu_sc as plsc

E = 8
K = 2
H = 1024
F = 1408
T = 2048

P = T * K
BM = 128
NB = P // BM + E
NBP = 64
P_pad = NB * BM

NC = 2
NS = 16
NW = NC * NS
TW = T // NW
TCH = 32

_NT = (((1,), (1,)), ((), ()))


def _lane_cumsum(x):
    pos = lax.broadcasted_iota(jnp.int32, x.shape, 1)
    sh = 1
    while sh < x.shape[1]:
        x = x + jnp.where(pos >= sh, jnp.roll(x, sh, axis=1), 0)
        sh *= 2
    return x


def _route_kernel(idsT_ref, p0_ref, p1_ref, bexp_ref):
    ids0 = idsT_ref[0:1, :]
    ids1 = idsT_ref[1:2, :]
    er = lax.broadcasted_iota(jnp.int32, (E, T), 0)
    oh0 = (ids0 == er).astype(jnp.int32)
    oh1 = (ids1 == er).astype(jnp.int32)
    c0 = _lane_cumsum(oh0)
    c1 = _lane_cumsum(oh1)
    cnt0 = c0[:, T - 1:T]
    counts = cnt0 + c1[:, T - 1:T]
    nblk = (counts + BM - 1) // BM

    seg_parts = [jnp.zeros((1, 1), jnp.int32)]
    for e in range(E - 1):
        seg_parts.append(seg_parts[-1] + nblk[e:e + 1, :])
    boff = jnp.concatenate(seg_parts, axis=0)
    seg = boff * BM

    p0_ref[...] = jnp.sum(jnp.where(oh0 != 0, seg + c0 - 1, 0),
                          axis=0, keepdims=True)
    p1_ref[...] = jnp.sum(jnp.where(oh1 != 0, seg + cnt0 + c1 - 1, 0),
                          axis=0, keepdims=True)
    jr = lax.broadcasted_iota(jnp.int32, (E, NBP), 1)
    clipped = jnp.clip(
        jnp.sum((jr >= boff).astype(jnp.int32), axis=0, keepdims=True) - 1,
        0, E - 1)
    tot = jnp.sum(nblk, axis=0, keepdims=True)
    jr2 = lax.broadcasted_iota(jnp.int32, (1, NBP), 1)
    bexp_ref[...] = jnp.where(jr2 == NB, tot, clipped)


def _route(idsT):
    return pl.pallas_call(
        _route_kernel,
        out_shape=(
            jax.ShapeDtypeStruct((1, T), jnp.int32),
            jax.ShapeDtypeStruct((1, T), jnp.int32),
            jax.ShapeDtypeStruct((1, NBP), jnp.int32),
        ),
    )(idsT)


@functools.lru_cache(maxsize=None)
def _sc_kernels():
    mesh = plsc.VectorSubcoreMesh(
        core_axis_name="c", subcore_axis_name="s",
        num_cores=NC, num_subcores=NS)

    @functools.partial(
        pl.kernel,
        out_type=jax.ShapeDtypeStruct((P_pad, H), jnp.float32),
        mesh=mesh,
        scratch_types=[
            pltpu.VMEM((TW, H), jnp.float32),
            pltpu.VMEM((TW,), jnp.int32),
            pltpu.VMEM((TW,), jnp.int32),
            pltpu.SemaphoreType.DMA,
        ],
    )
    def sc_dispatch(x_hbm, p0_hbm, p1_hbm, out_hbm, xl_v, i0_v, i1_v, sem):
        wid = lax.axis_index("s") * NC + lax.axis_index("c")
        base = wid * TW
        pltpu.sync_copy(x_hbm.at[pl.ds(base, TW)], xl_v)
        pltpu.sync_copy(p0_hbm.at[0, pl.ds(base, TW)], i0_v)
        pltpu.sync_copy(p1_hbm.at[0, pl.ds(base, TW)], i1_v)
        d0 = pltpu.make_async_copy(xl_v, out_hbm.at[i0_v], sem)
        d1 = pltpu.make_async_copy(xl_v, out_hbm.at[i1_v], sem)
        d0.start()
        d1.start()
        d0.wait()
        d1.wait()

    @functools.partial(
        pl.kernel,
        out_type=(jax.ShapeDtypeStruct((T, H), jnp.float32),
                  jax.ShapeDtypeStruct((T, H), jnp.float32)),
        mesh=mesh,
        scratch_types=[
            pltpu.VMEM((TCH,), jnp.int32),
            pltpu.VMEM((TCH,), jnp.int32),
            pltpu.VMEM((TCH, H), jnp.float32),
            pltpu.VMEM((TCH, H), jnp.float32),
            pltpu.SemaphoreType.DMA,
        ],
    )
    def sc_combine(ys_hbm, p0_hbm, p1_hbm, a_hbm, b_hbm,
                   i0_v, i1_v, a_v, b_v, sem):
        wid = lax.axis_index("s") * NC + lax.axis_index("c")
        base = wid * TW
        for c in range(TW // TCH):
            off = base + c * TCH
            pltpu.sync_copy(p0_hbm.at[0, pl.ds(off, TCH)], i0_v)
            pltpu.sync_copy(p1_hbm.at[0, pl.ds(off, TCH)], i1_v)
            d0 = pltpu.make_async_copy(ys_hbm.at[i0_v], a_v, sem)
            d1 = pltpu.make_async_copy(ys_hbm.at[i1_v], b_v, sem)
            d0.start()
            d1.start()
            d0.wait()
            d1.wait()
            pltpu.sync_copy(a_v, a_hbm.at[pl.ds(off, TCH)])
            pltpu.sync_copy(b_v, b_hbm.at[pl.ds(off, TCH)])

    return sc_dispatch, sc_combine


def _ffn_kernel(bexp_ref, xg_ref, gu_ref, dn_ref, ys_ref, gub, dnb):
    b = pl.program_id(0)
    used = b < bexp_ref[0, NB]
    prev = bexp_ref[0, jnp.maximum(b - 1, 0)]
    changed = jnp.logical_and(
        used, jnp.logical_or(b == 0, bexp_ref[0, b] != prev))

    @pl.when(changed)
    def _cast():
        gub[...] = gu_ref[0].astype(jnp.bfloat16)
        dnb[...] = dn_ref[0].astype(jnp.bfloat16)

    @pl.when(used)
    def _compute():
        xg = xg_ref[...].astype(jnp.bfloat16)
        hg = lax.dot_general(xg, gub[0], _NT,
                             preferred_element_type=jnp.float32)
        hu = lax.dot_general(xg, gub[1], _NT,
                             preferred_element_type=jnp.float32)
        act = (jax.nn.silu(hg) * hu).astype(jnp.bfloat16)
        ys_ref[...] = lax.dot_general(act, dnb[...], _NT,
                                      preferred_element_type=jnp.float32)


def _grouped_ffn(bexp, xg, gu4, dn):
    grid_spec = pltpu.PrefetchScalarGridSpec(
        num_scalar_prefetch=1,
        grid=(NB,),
        in_specs=[
            pl.BlockSpec((BM, H), lambda b, bexp: (b, 0)),
            pl.BlockSpec((1, 2, F, H), lambda b, bexp: (bexp[0, b], 0, 0, 0)),
            pl.BlockSpec((1, H, F), lambda b, bexp: (bexp[0, b], 0, 0)),
        ],
        out_specs=pl.BlockSpec((BM, H), lambda b, bexp: (b, 0)),
        scratch_shapes=[
            pltpu.VMEM((2, F, H), jnp.bfloat16),
            pltpu.VMEM((H, F), jnp.bfloat16),
        ],
    )
    return pl.pallas_call(
        _ffn_kernel,
        grid_spec=grid_spec,
        out_shape=jax.ShapeDtypeStruct((P_pad, H), jnp.float32),
    )(bexp, xg, gu4, dn)


CTM = 256


def _wadd_kernel(a_ref, b_ref, w0_ref, w1_ref, out_ref):
    out_ref[...] = a_ref[...] * w0_ref[...] + b_ref[...] * w1_ref[...]


def _wadd(a, b, w0, w1):
    return pl.pallas_call(
        _wadd_kernel,
        grid=(T // CTM,),
        in_specs=[
            pl.BlockSpec((CTM, H), lambda t: (t, 0)),
            pl.BlockSpec((CTM, H), lambda t: (t, 0)),
            pl.BlockSpec((CTM, 1), lambda t: (t, 0)),
            pl.BlockSpec((CTM, 1), lambda t: (t, 0)),
        ],
        out_specs=pl.BlockSpec((CTM, H), lambda t: (t, 0)),
        out_shape=jax.ShapeDtypeStruct((T, H), jnp.float32),
    )(a, b, w0, w1)


@jax.jit
def kernel(x, topk_ids, topk_weight, gate_up_weights, down_weights):
    idsT = topk_ids.astype(jnp.int32).T
    p0, p1, bexp = _route(idsT)

    sc_dispatch, sc_combine = _sc_kernels()
    xg = sc_dispatch(x, p0, p1)

    gu4 = gate_up_weights.reshape(E, 2, F, H)
    ys = _grouped_ffn(bexp, xg, gu4, down_weights)

    a, b = sc_combine(ys, p0, p1)
    return _wadd(a, b, topk_weight[:, 0:1], topk_weight[:, 1:2])


# speedup vs baseline: 1.3236x; 1.3236x over previous
import functools

import jax
import jax.numpy as jnp
from jax import lax
from jax.experimental import pallas as pl
from jax.experimental.pallas import tpu as pltpu
from jax.experimental.pallas import tpu_sc as plsc

E = 8
K = 2
H = 1024
F = 1408
T = 2048

P = T * K
BM = 256
NB = P // BM + E
NBP = 64
P_pad = NB * BM

NC = 2
NS = 16
NW = NC * NS
TW = T // NW
TCH = 32

_NT = (((1,), (1,)), ((), ()))


def _lane_cumsum(x):
    pos = lax.broadcasted_iota(jnp.int32, x.shape, 1)
    sh = 1
    while sh < x.shape[1]:
        x = x + jnp.where(pos >= sh, jnp.roll(x, sh, axis=1), 0)
        sh *= 2
    return x


def _route_kernel(idsT_ref, p0_ref, p1_ref, bexp_ref):
    ids0 = idsT_ref[0:1, :]
    ids1 = idsT_ref[1:2, :]
    er = lax.broadcasted_iota(jnp.int32, (E, T), 0)
    oh0 = (ids0 == er).astype(jnp.int32)
    oh1 = (ids1 == er).astype(jnp.int32)
    c0 = _lane_cumsum(oh0)
    c1 = _lane_cumsum(oh1)
    cnt0 = c0[:, T - 1:T]
    counts = cnt0 + c1[:, T - 1:T]
    nblk = (counts + BM - 1) // BM

    seg_parts = [jnp.zeros((1, 1), jnp.int32)]
    for e in range(E - 1):
        seg_parts.append(seg_parts[-1] + nblk[e:e + 1, :])
    boff = jnp.concatenate(seg_parts, axis=0)
    seg = boff * BM

    p0_ref[...] = jnp.sum(jnp.where(oh0 != 0, seg + c0 - 1, 0),
                          axis=0, keepdims=True)
    p1_ref[...] = jnp.sum(jnp.where(oh1 != 0, seg + cnt0 + c1 - 1, 0),
                          axis=0, keepdims=True)
    jr = lax.broadcasted_iota(jnp.int32, (E, NBP), 1)
    clipped = jnp.clip(
        jnp.sum((jr >= boff).astype(jnp.int32), axis=0, keepdims=True) - 1,
        0, E - 1)
    tot = jnp.sum(nblk, axis=0, keepdims=True)
    jr2 = lax.broadcasted_iota(jnp.int32, (1, NBP), 1)
    bexp_ref[...] = jnp.where(jr2 == NB, tot, clipped)


def _route(idsT):
    return pl.pallas_call(
        _route_kernel,
        out_shape=(
            jax.ShapeDtypeStruct((1, T), jnp.int32),
            jax.ShapeDtypeStruct((1, T), jnp.int32),
            jax.ShapeDtypeStruct((1, NBP), jnp.int32),
        ),
    )(idsT)


@functools.lru_cache(maxsize=None)
def _sc_kernels():
    mesh = plsc.VectorSubcoreMesh(
        core_axis_name="c", subcore_axis_name="s",
        num_cores=NC, num_subcores=NS)

    @functools.partial(
        pl.kernel,
        out_type=jax.ShapeDtypeStruct((P_pad, H), jnp.float32),
        mesh=mesh,
        scratch_types=[
            pltpu.VMEM((TW, H), jnp.float32),
            pltpu.VMEM((TW,), jnp.int32),
            pltpu.VMEM((TW,), jnp.int32),
            pltpu.SemaphoreType.DMA,
        ],
    )
    def sc_dispatch(x_hbm, p0_hbm, p1_hbm, out_hbm, xl_v, i0_v, i1_v, sem):
        wid = lax.axis_index("s") * NC + lax.axis_index("c")
        base = wid * TW
        pltpu.sync_copy(x_hbm.at[pl.ds(base, TW)], xl_v)
        pltpu.sync_copy(p0_hbm.at[0, pl.ds(base, TW)], i0_v)
        pltpu.sync_copy(p1_hbm.at[0, pl.ds(base, TW)], i1_v)
        d0 = pltpu.make_async_copy(xl_v, out_hbm.at[i0_v], sem)
        d1 = pltpu.make_async_copy(xl_v, out_hbm.at[i1_v], sem)
        d0.start()
        d1.start()
        d0.wait()
        d1.wait()

    @functools.partial(
        pl.kernel,
        out_type=(jax.ShapeDtypeStruct((T, H), jnp.float32),
                  jax.ShapeDtypeStruct((T, H), jnp.float32)),
        mesh=mesh,
        scratch_types=[
            pltpu.VMEM((TCH,), jnp.int32),
            pltpu.VMEM((TCH,), jnp.int32),
            pltpu.VMEM((TCH, H), jnp.float32),
            pltpu.VMEM((TCH, H), jnp.float32),
            pltpu.SemaphoreType.DMA,
        ],
    )
    def sc_combine(ys_hbm, p0_hbm, p1_hbm, a_hbm, b_hbm,
                   i0_v, i1_v, a_v, b_v, sem):
        wid = lax.axis_index("s") * NC + lax.axis_index("c")
        base = wid * TW
        for c in range(TW // TCH):
            off = base + c * TCH
            pltpu.sync_copy(p0_hbm.at[0, pl.ds(off, TCH)], i0_v)
            pltpu.sync_copy(p1_hbm.at[0, pl.ds(off, TCH)], i1_v)
            d0 = pltpu.make_async_copy(ys_hbm.at[i0_v], a_v, sem)
            d1 = pltpu.make_async_copy(ys_hbm.at[i1_v], b_v, sem)
            d0.start()
            d1.start()
            d0.wait()
            d1.wait()
            pltpu.sync_copy(a_v, a_hbm.at[pl.ds(off, TCH)])
            pltpu.sync_copy(b_v, b_hbm.at[pl.ds(off, TCH)])

    return sc_dispatch, sc_combine


def _ffn_kernel(bexp_ref, xg_ref, gu_ref, dn_ref, ys_ref, gub, dnb):
    b = pl.program_id(0)
    used = b < bexp_ref[0, NB]
    prev = bexp_ref[0, jnp.maximum(b - 1, 0)]
    changed = jnp.logical_and(
        used, jnp.logical_or(b == 0, bexp_ref[0, b] != prev))

    @pl.when(changed)
    def _cast():
        gub[...] = gu_ref[0].astype(jnp.bfloat16)
        dnb[...] = dn_ref[0].astype(jnp.bfloat16)

    @pl.when(used)
    def _compute():
        xg = xg_ref[...].astype(jnp.bfloat16)
        hg = lax.dot_general(xg, gub[0], _NT,
                             preferred_element_type=jnp.float32)
        hu = lax.dot_general(xg, gub[1], _NT,
                             preferred_element_type=jnp.float32)
        act = (jax.nn.silu(hg) * hu).astype(jnp.bfloat16)
        ys_ref[...] = lax.dot_general(act, dnb[...], _NT,
                                      preferred_element_type=jnp.float32)


def _grouped_ffn(bexp, xg, gu4, dn):
    grid_spec = pltpu.PrefetchScalarGridSpec(
        num_scalar_prefetch=1,
        grid=(NB,),
        in_specs=[
            pl.BlockSpec((BM, H), lambda b, bexp: (b, 0)),
            pl.BlockSpec((1, 2, F, H), lambda b, bexp: (bexp[0, b], 0, 0, 0)),
            pl.BlockSpec((1, H, F), lambda b, bexp: (bexp[0, b], 0, 0)),
        ],
        out_specs=pl.BlockSpec((BM, H), lambda b, bexp: (b, 0)),
        scratch_shapes=[
            pltpu.VMEM((2, F, H), jnp.bfloat16),
            pltpu.VMEM((H, F), jnp.bfloat16),
        ],
    )
    return pl.pallas_call(
        _ffn_kernel,
        grid_spec=grid_spec,
        out_shape=jax.ShapeDtypeStruct((P_pad, H), jnp.float32),
    )(bexp, xg, gu4, dn)


CTM = 256


def _wadd_kernel(a_ref, b_ref, w0_ref, w1_ref, out_ref):
    out_ref[...] = a_ref[...] * w0_ref[...] + b_ref[...] * w1_ref[...]


def _wadd(a, b, w0, w1):
    return pl.pallas_call(
        _wadd_kernel,
        grid=(T // CTM,),
        in_specs=[
            pl.BlockSpec((CTM, H), lambda t: (t, 0)),
            pl.BlockSpec((CTM, H), lambda t: (t, 0)),
            pl.BlockSpec((CTM, 1), lambda t: (t, 0)),
            pl.BlockSpec((CTM, 1), lambda t: (t, 0)),
        ],
        out_specs=pl.BlockSpec((CTM, H), lambda t: (t, 0)),
        out_shape=jax.ShapeDtypeStruct((T, H), jnp.float32),
    )(a, b, w0, w1)


@jax.jit
def kernel(x, topk_ids, topk_weight, gate_up_weights, down_weights):
    idsT = topk_ids.astype(jnp.int32).T
    p0, p1, bexp = _route(idsT)

    sc_dispatch, sc_combine = _sc_kernels()
    xg = sc_dispatch(x, p0, p1)

    gu4 = gate_up_weights.reshape(E, 2, F, H)
    ys = _grouped_ffn(bexp, xg, gu4, down_weights)

    a, b = sc_combine(ys, p0, p1)
    return _wadd(a, b, topk_weight[:, 0:1], topk_weight[:, 1:2])
